# baseline (speedup 1.0000x reference)
import functools

import jax
import jax.numpy as jnp
from jax import lax
from jax.experimental import pallas as pl
from jax.experimental.pallas import tpu as pltpu
from jax.experimental.pallas import tpu_sc as plsc

N_TOKENS = 128
D_MODEL = 768
N_EXPERTS = 16
D_FF = 3072
F_TILE = 1536
F_TILES = D_FF // F_TILE

_NEG = -1e30
_ROWS_PER_SUBCORE = 8


def _logits_body(x_ref, wg_ref, out_ref):
    out_ref[...] = jnp.dot(
        x_ref[...], wg_ref[...], preferred_element_type=jnp.float32
    )


def _shuffle(v, idx):
    return v.at[idx].get(mode="promise_in_bounds")


def _bfly_max(v, io):
    for sh in (8, 4, 2, 1):
        v = jnp.maximum(v, _shuffle(v, jnp.bitwise_xor(io, sh)))
    return v


def _bfly_min(v, io):
    for sh in (8, 4, 2, 1):
        v = jnp.minimum(v, _shuffle(v, jnp.bitwise_xor(io, sh)))
    return v


def _sc_combine_body(logits_hbm, out_hbm, lbuf, cbuf):
    sid = lax.axis_index("s")
    cid = lax.axis_index("c")

    @pl.when(cid == 0)
    def _work():
        base = sid * _ROWS_PER_SUBCORE
        pltpu.sync_copy(logits_hbm.at[pl.ds(base, _ROWS_PER_SUBCORE)], lbuf)
        io = lax.iota(jnp.int32, 16)
        for i in range(_ROWS_PER_SUBCORE):
            l = lbuf[i, :]
            m1 = _bfly_max(l, io)
            a1 = _bfly_min(jnp.where(l == m1, io, N_EXPERTS), io)
            masked = jnp.where(io == a1, _NEG, l)
            m2 = _bfly_max(masked, io)
            a2 = _bfly_min(jnp.where(masked == m2, io, N_EXPERTS), io)
            ev = jnp.exp(m2 - m1)
            w1v = 1.0 / (1.0 + ev)
            cbuf[i, :] = jnp.where(io == a1, w1v, 0.0) + jnp.where(
                io == a2, 1.0 - w1v, 0.0
            )
        pltpu.sync_copy(cbuf, out_hbm.at[pl.ds(base, _ROWS_PER_SUBCORE)])


@functools.partial(
    pl.kernel,
    mesh=plsc.VectorSubcoreMesh(core_axis_name="c", subcore_axis_name="s"),
    out_type=jax.ShapeDtypeStruct((N_TOKENS, N_EXPERTS), jnp.float32),
    scratch_types=[
        pltpu.VMEM((_ROWS_PER_SUBCORE, N_EXPERTS), jnp.float32),
        pltpu.VMEM((_ROWS_PER_SUBCORE, N_EXPERTS), jnp.float32),
    ],
)
def _sc_combine(logits_hbm, out_hbm, lbuf, cbuf):
    _sc_combine_body(logits_hbm, out_hbm, lbuf, cbuf)


def _ffn_body(x_ref, comb_ref, w1_ref, w2_ref, out_ref):
    e = pl.program_id(0)
    f = pl.program_id(1)
    lane = jax.lax.broadcasted_iota(jnp.int32, (N_TOKENS, N_EXPERTS), 1)
    ce = jnp.sum(
        jnp.where(lane == e, comb_ref[...], 0.0), axis=1, keepdims=True
    )
    h = jnp.dot(x_ref[...], w1_ref[0], preferred_element_type=jnp.float32)
    h = 0.5 * h * (1.0 + jax.lax.erf(h * 0.7071067811865476))
    part = jnp.dot(h * ce, w2_ref[0], preferred_element_type=jnp.float32)

    @pl.when(jnp.logical_and(e == 0, f == 0))
    def _first():
        out_ref[...] = part

    @pl.when(jnp.logical_or(e > 0, f > 0))
    def _rest():
        out_ref[...] += part


@jax.jit
def kernel(x, Wg, W1, W2):
    logits = pl.pallas_call(
        _logits_body,
        in_specs=[
            pl.BlockSpec((N_TOKENS, D_MODEL), lambda: (0, 0)),
            pl.BlockSpec((D_MODEL, N_EXPERTS), lambda: (0, 0)),
        ],
        out_specs=pl.BlockSpec((N_TOKENS, N_EXPERTS), lambda: (0, 0)),
        out_shape=jax.ShapeDtypeStruct((N_TOKENS, N_EXPERTS), jnp.float32),
    )(x, Wg)

    combine = _sc_combine(logits)

    return pl.pallas_call(
        _ffn_body,
        grid=(N_EXPERTS, F_TILES),
        in_specs=[
            pl.BlockSpec((N_TOKENS, D_MODEL), lambda e, f: (0, 0)),
            pl.BlockSpec((N_TOKENS, N_EXPERTS), lambda e, f: (0, 0)),
            pl.BlockSpec((1, D_MODEL, F_TILE), lambda e, f: (e, 0, f)),
            pl.BlockSpec((1, F_TILE, D_MODEL), lambda e, f: (e, f, 0)),
        ],
        out_specs=pl.BlockSpec((N_TOKENS, D_MODEL), lambda e, f: (0, 0)),
        out_shape=jax.ShapeDtypeStruct((N_TOKENS, D_MODEL), jnp.float32),
        compiler_params=pltpu.CompilerParams(
            dimension_semantics=("arbitrary", "arbitrary"),
        ),
    )(x, combine, W1, W2)


# speedup vs baseline: 1.0529x; 1.0529x over previous
import jax
import jax.numpy as jnp
from jax.experimental import pallas as pl
from jax.experimental.pallas import tpu as pltpu

N_TOKENS = 128
D_MODEL = 768
N_EXPERTS = 16
D_FF = 3072
F_TILE = 1536
F_TILES = D_FF // F_TILE

_NEG = -1e30


def _moe_body(x_ref, wg_ref, w1_ref, w2_ref, out_ref, combine_ref):
    e = pl.program_id(0)
    f = pl.program_id(1)

    @pl.when(jnp.logical_and(e == 0, f == 0))
    def _init():
        x = x_ref[...]
        logits = jnp.dot(x, wg_ref[...], preferred_element_type=jnp.float32)
        lane = jax.lax.broadcasted_iota(jnp.int32, (N_TOKENS, N_EXPERTS), 1)
        m1 = jnp.max(logits, axis=1, keepdims=True)
        cand1 = jnp.where(logits == m1, lane, N_EXPERTS)
        a1 = jnp.min(cand1, axis=1, keepdims=True)
        oh1 = (lane == a1).astype(jnp.float32)
        masked = jnp.where(lane == a1, _NEG, logits)
        m2 = jnp.max(masked, axis=1, keepdims=True)
        cand2 = jnp.where(masked == m2, lane, N_EXPERTS)
        a2 = jnp.min(cand2, axis=1, keepdims=True)
        oh2 = (lane == a2).astype(jnp.float32)
        w_first = 1.0 / (1.0 + jnp.exp(m2 - m1))
        combine_ref[...] = w_first * oh1 + (1.0 - w_first) * oh2

    lane = jax.lax.broadcasted_iota(jnp.int32, (N_TOKENS, N_EXPERTS), 1)
    ce = jnp.sum(
        jnp.where(lane == e, combine_ref[...], 0.0), axis=1, keepdims=True
    )
    h = jnp.dot(x_ref[...], w1_ref[0], preferred_element_type=jnp.float32)
    h = 0.5 * h * (1.0 + jax.lax.erf(h * 0.7071067811865476))
    w2f = w2_ref[0, pl.ds(f * F_TILE, F_TILE), :]
    part = jnp.dot(h * ce, w2f, preferred_element_type=jnp.float32)

    @pl.when(jnp.logical_and(e == 0, f == 0))
    def _first():
        out_ref[...] = part

    @pl.when(jnp.logical_or(e > 0, f > 0))
    def _rest():
        out_ref[...] += part


@jax.jit
def kernel(x, Wg, W1, W2):
    return pl.pallas_call(
        _moe_body,
        grid=(N_EXPERTS, F_TILES),
        in_specs=[
            pl.BlockSpec((N_TOKENS, D_MODEL), lambda e, f: (0, 0)),
            pl.BlockSpec((D_MODEL, N_EXPERTS), lambda e, f: (0, 0)),
            pl.BlockSpec((1, D_MODEL, F_TILE), lambda e, f: (e, 0, f)),
            pl.BlockSpec((1, D_FF, D_MODEL), lambda e, f: (e, 0, 0)),
        ],
        out_specs=pl.BlockSpec((N_TOKENS, D_MODEL), lambda e, f: (0, 0)),
        out_shape=jax.ShapeDtypeStruct((N_TOKENS, D_MODEL), jnp.float32),
        scratch_shapes=[pltpu.VMEM((N_TOKENS, N_EXPERTS), jnp.float32)],
        compiler_params=pltpu.CompilerParams(
            dimension_semantics=("arbitrary", "arbitrary"),
        ),
    )(x, Wg, W1, W2)


# speedup vs baseline: 1.1947x; 1.1346x over previous
import jax
import jax.numpy as jnp
from jax.experimental import pallas as pl
from jax.experimental.pallas import tpu as pltpu

N_TOKENS = 128
D_MODEL = 768
N_EXPERTS = 16
D_FF = 3072
F_TILE = 1536
F_TILES = D_FF // F_TILE

_NEG = -1e30


def _moe_body(x_ref, wg_ref, w1_ref, w2_ref, out_ref, combine_ref):
    e = pl.program_id(0)
    f = pl.program_id(1)

    @pl.when(jnp.logical_and(e == 0, f == 0))
    def _init():
        x = x_ref[...]
        logits = jnp.dot(x, wg_ref[...], preferred_element_type=jnp.float32)
        lane = jax.lax.broadcasted_iota(jnp.int32, (N_TOKENS, N_EXPERTS), 1)
        m1 = jnp.max(logits, axis=1, keepdims=True)
        cand1 = jnp.where(logits == m1, lane, N_EXPERTS)
        a1 = jnp.min(cand1, axis=1, keepdims=True)
        oh1 = (lane == a1).astype(jnp.float32)
        masked = jnp.where(lane == a1, _NEG, logits)
        m2 = jnp.max(masked, axis=1, keepdims=True)
        cand2 = jnp.where(masked == m2, lane, N_EXPERTS)
        a2 = jnp.min(cand2, axis=1, keepdims=True)
        oh2 = (lane == a2).astype(jnp.float32)
        w_first = 1.0 / (1.0 + jnp.exp(m2 - m1))
        combine_ref[...] = w_first * oh1 + (1.0 - w_first) * oh2

    lane = jax.lax.broadcasted_iota(jnp.int32, (N_TOKENS, N_EXPERTS), 1)
    ce = jnp.sum(
        jnp.where(lane == e, combine_ref[...], 0.0), axis=1, keepdims=True
    )
    h = jnp.dot(x_ref[...], w1_ref[0], preferred_element_type=jnp.float32)
    h = 0.5 * h * (1.0 + jax.lax.erf(h * 0.7071067811865476))
    part = jnp.dot(h * ce, w2_ref[0], preferred_element_type=jnp.float32)

    @pl.when(jnp.logical_and(e == 0, f == 0))
    def _first():
        out_ref[...] = part

    @pl.when(jnp.logical_or(e > 0, f > 0))
    def _rest():
        out_ref[...] += part


@jax.jit
def kernel(x, Wg, W1, W2):
    return pl.pallas_call(
        _moe_body,
        grid=(N_EXPERTS, F_TILES),
        in_specs=[
            pl.BlockSpec((N_TOKENS, D_MODEL), lambda e, f: (0, 0)),
            pl.BlockSpec((D_MODEL, N_EXPERTS), lambda e, f: (0, 0)),
            pl.BlockSpec((1, D_MODEL, F_TILE), lambda e, f: (e, 0, f)),
            pl.BlockSpec((1, F_TILE, D_MODEL), lambda e, f: (e, f, 0)),
        ],
        out_specs=pl.BlockSpec((N_TOKENS, D_MODEL), lambda e, f: (0, 0)),
        out_shape=jax.ShapeDtypeStruct((N_TOKENS, D_MODEL), jnp.float32),
        scratch_shapes=[pltpu.VMEM((N_TOKENS, N_EXPERTS), jnp.float32)],
        compiler_params=pltpu.CompilerParams(
            dimension_semantics=("arbitrary", "arbitrary"),
        ),
    )(x, Wg, W1, W2)
